# Initial kernel scaffold; baseline (speedup 1.0000x reference)
import jax
import jax.numpy as jnp
from jax.experimental import pallas as pl


def kernel(x_p, x_a, adj_p_a, adj_a_p, W_rel_p_a, W_rel_a_p, Wcat_p, bcat_p, Wcat_a, bcat_a, Wff_p, bff_p, Wff_a, bff_a, g_hn_p, g_hn_a, g_fn_p, g_fn_a, b_hn_p, b_hn_a, b_fn_p, b_fn_a):
    raise NotImplementedError("write your pallas kernel here")


# speedup vs baseline: 1.1432x; 1.1432x over previous
import functools

import jax
import jax.numpy as jnp
from jax.experimental import pallas as pl
from jax.experimental.pallas import tpu as pltpu


def _layernorm(x, g, b, eps=1e-5):
    m = jnp.mean(x, axis=-1, keepdims=True)
    xc = x - m
    v = jnp.mean(xc * xc, axis=-1, keepdims=True)
    return xc * jax.lax.rsqrt(v + eps) * g + b


def _fused_block_kernel(adj_ref, xsrc_ref, xdst_ref, wrel_ref, wn_ref, ws_ref,
                        bcat_ref, wff_ref, bff_ref, ghn_ref, bhn_ref,
                        gfn_ref, bfn_ref, out_ref):
    acc = jnp.dot(adj_ref[...].astype(jnp.bfloat16), xsrc_ref[...],
                  preferred_element_type=jnp.float32)
    nb = jnp.dot(acc, wrel_ref[...], preferred_element_type=jnp.float32)
    x = xdst_ref[...]
    out = (jnp.dot(nb, wn_ref[...], preferred_element_type=jnp.float32)
           + jnp.dot(x, ws_ref[...], preferred_element_type=jnp.float32)
           + bcat_ref[...])
    y = _layernorm(out + x, ghn_ref[...], bhn_ref[...])
    z = jax.nn.relu(jnp.dot(y, wff_ref[...], preferred_element_type=jnp.float32)
                    + bff_ref[...])
    out_ref[...] = _layernorm(z + y, gfn_ref[...], bfn_ref[...])


@functools.partial(jax.jit, static_argnames=("block_rows",))
def _hete_block(adj, x_src_bf16, x_dst, w_rel, wn, ws, bcat, wff, bff,
                g_hn, b_hn, g_fn, b_fn, block_rows=400):
    m, n = adj.shape
    d = x_dst.shape[1]
    br = min(block_rows, m)
    grid = (pl.cdiv(m, br),)
    row2 = lambda i: (i, 0)
    full = lambda i: (0, 0)
    vec_spec = pl.BlockSpec((1, d), full)
    mat_spec = pl.BlockSpec((d, d), full)
    return pl.pallas_call(
        _fused_block_kernel,
        grid=grid,
        in_specs=[
            pl.BlockSpec((br, n), row2),
            pl.BlockSpec((n, d), full),
            pl.BlockSpec((br, d), row2),
            mat_spec, mat_spec, mat_spec,
            vec_spec,
            mat_spec, vec_spec,
            vec_spec, vec_spec, vec_spec, vec_spec,
        ],
        out_specs=pl.BlockSpec((br, d), row2),
        out_shape=jax.ShapeDtypeStruct((m, d), jnp.float32),
        compiler_params=pltpu.CompilerParams(
            dimension_semantics=("arbitrary",)),
    )(adj, x_src_bf16, x_dst, w_rel, wn, ws, bcat, wff, bff,
      g_hn, b_hn, g_fn, b_fn)


def kernel(x_p, x_a, adj_p_a, adj_a_p, W_rel_p_a, W_rel_a_p, Wcat_p, bcat_p,
           Wcat_a, bcat_a, Wff_p, bff_p, Wff_a, bff_a, g_hn_p, g_hn_a,
           g_fn_p, g_fn_a, b_hn_p, b_hn_a, b_fn_p, b_fn_a):
    d = x_p.shape[1]
    row = lambda v: v.reshape(1, d)
    z_p = _hete_block(
        adj_p_a, x_a.astype(jnp.bfloat16), x_p, W_rel_p_a,
        Wcat_p[:, :d].T, Wcat_p[:, d:].T, row(bcat_p),
        Wff_p.T, row(bff_p), row(g_hn_p), row(b_hn_p), row(g_fn_p),
        row(b_fn_p))
    z_a = _hete_block(
        adj_a_p, x_p.astype(jnp.bfloat16), x_a, W_rel_a_p,
        Wcat_a[:, :d].T, Wcat_a[:, d:].T, row(bcat_a),
        Wff_a.T, row(bff_a), row(g_hn_a), row(b_hn_a), row(g_fn_a),
        row(b_fn_a))
    return (z_p, z_a)


# speedup vs baseline: 1.1692x; 1.0227x over previous
import functools

import jax
import jax.numpy as jnp
from jax.experimental import pallas as pl
from jax.experimental.pallas import tpu as pltpu


def _layernorm(x, g, b, eps=1e-5):
    m = jnp.mean(x, axis=-1, keepdims=True)
    xc = x - m
    v = jnp.mean(xc * xc, axis=-1, keepdims=True)
    return xc * jax.lax.rsqrt(v + eps) * g + b


def _fused_block_kernel(adjt_ref, adjb_ref, xsrc_ref, xdst_ref, wrel_ref,
                        wn_ref, ws_ref, bcat_ref, wff_ref, bff_ref, ghn_ref,
                        bhn_ref, gfn_ref, bfn_ref, out_ref):
    xsrc = xsrc_ref[...]
    acc = jnp.concatenate(
        [jnp.dot(adjt_ref[...].astype(jnp.bfloat16), xsrc,
                 preferred_element_type=jnp.float32),
         jnp.dot(adjb_ref[...].astype(jnp.bfloat16), xsrc,
                 preferred_element_type=jnp.float32)], axis=0)
    nb = jnp.dot(acc, wrel_ref[...], preferred_element_type=jnp.float32)
    x = xdst_ref[...]
    out = (jnp.dot(nb, wn_ref[...], preferred_element_type=jnp.float32)
           + jnp.dot(x, ws_ref[...], preferred_element_type=jnp.float32)
           + bcat_ref[...])
    y = _layernorm(out + x, ghn_ref[...], bhn_ref[...])
    z = jax.nn.relu(jnp.dot(y, wff_ref[...], preferred_element_type=jnp.float32)
                    + bff_ref[...])
    out_ref[...] = _layernorm(z + y, gfn_ref[...], bfn_ref[...])


@functools.partial(jax.jit, static_argnames=("block_rows",))
def _hete_block(adj, x_src_bf16, x_dst, w_rel, wn, ws, bcat, wff, bff,
                g_hn, b_hn, g_fn, b_fn, block_rows=400):
    m, n = adj.shape
    d = x_dst.shape[1]
    br = min(block_rows, m)
    brh = br // 2
    grid = (pl.cdiv(m, br),)
    row2 = lambda i: (i, 0)
    top = lambda i: (2 * i, 0)
    bot = lambda i: (2 * i + 1, 0)
    full = lambda i: (0, 0)
    vec_spec = pl.BlockSpec((1, d), full)
    mat_spec = pl.BlockSpec((d, d), full)
    return pl.pallas_call(
        _fused_block_kernel,
        grid=grid,
        in_specs=[
            pl.BlockSpec((brh, n), top),
            pl.BlockSpec((brh, n), bot),
            pl.BlockSpec((n, d), full),
            pl.BlockSpec((br, d), row2),
            mat_spec, mat_spec, mat_spec,
            vec_spec,
            mat_spec, vec_spec,
            vec_spec, vec_spec, vec_spec, vec_spec,
        ],
        out_specs=pl.BlockSpec((br, d), row2),
        out_shape=jax.ShapeDtypeStruct((m, d), jnp.float32),
        compiler_params=pltpu.CompilerParams(
            dimension_semantics=("arbitrary",)),
    )(adj, adj, x_src_bf16, x_dst, w_rel, wn, ws, bcat, wff, bff,
      g_hn, b_hn, g_fn, b_fn)


def kernel(x_p, x_a, adj_p_a, adj_a_p, W_rel_p_a, W_rel_a_p, Wcat_p, bcat_p,
           Wcat_a, bcat_a, Wff_p, bff_p, Wff_a, bff_a, g_hn_p, g_hn_a,
           g_fn_p, g_fn_a, b_hn_p, b_hn_a, b_fn_p, b_fn_a):
    d = x_p.shape[1]
    row = lambda v: v.reshape(1, d)
    z_p = _hete_block(
        adj_p_a, x_a.astype(jnp.bfloat16), x_p, W_rel_p_a,
        Wcat_p[:, :d].T, Wcat_p[:, d:].T, row(bcat_p),
        Wff_p.T, row(bff_p), row(g_hn_p), row(b_hn_p), row(g_fn_p),
        row(b_fn_p))
    z_a = _hete_block(
        adj_a_p, x_p.astype(jnp.bfloat16), x_a, W_rel_a_p,
        Wcat_a[:, :d].T, Wcat_a[:, d:].T, row(bcat_a),
        Wff_a.T, row(bff_a), row(g_hn_a), row(b_hn_a), row(g_fn_a),
        row(b_fn_a))
    return (z_p, z_a)
